# Initial kernel scaffold; baseline (speedup 1.0000x reference)
import jax
import jax.numpy as jnp
from jax.experimental import pallas as pl


def kernel(x_job, x_worker, edge_precede, edge_next, proc_src, proc_dst, W_emb_job, b_emb_job, W_emb_worker, b_emb_worker, W_prec, b_prec, W_next, b_next, W_self, W_neigh, b_sage):
    raise NotImplementedError("write your pallas kernel here")


# speedup vs baseline: 5.9313x; 5.9313x over previous
import functools

import jax
import jax.numpy as jnp
from jax import lax
from jax.experimental import pallas as pl
from jax.experimental.pallas import tpu as pltpu
from jax.experimental.pallas import tpu_sc as plsc

N_JOB = 100000
N_WORKER = 10000
E = 1600000
D = 16
NUM_LOOPS = 2

NC = 2
NS = 16
LANES = 128
EROWS = E // LANES
EROWS_P = 12544
EPAD = EROWS_P * LANES - E
CH = 8
CHD = 8
ST = 6256

_MESH = plsc.VectorSubcoreMesh(core_axis_name="c", subcore_axis_name="s")

f32 = jnp.float32


def _striped(fn, s):
  @pl.when(s < NS - 1)
  def _():
    fn(s * ST, ST)

  @pl.when(s == NS - 1)
  def _():
    fn((NS - 1) * ST, N_JOB - (NS - 1) * ST)


def _sc_degrees(eps, epd, ens, end_, pcd, z16, ones16):
  @functools.partial(
      pl.kernel,
      mesh=_MESH,
      compiler_params=pltpu.CompilerParams(use_tc_tiling_on_sc=False),
      out_type=jax.ShapeDtypeStruct((NC, 5, N_JOB, D), f32),
      scratch_types=[
          pltpu.VMEM((CH, LANES), jnp.int32),
          pltpu.VMEM((LANES, D), f32),
          pltpu.VMEM_SHARED((N_JOB + 1, D), f32),
          pltpu.SemaphoreType.DMA,
      ],
  )
  def k(eps_h, epd_h, ens_h, end_h, pcd_h, z16_h, ones_h, out_h,
        idx_v, ones_v, acc, sem):
    c = lax.axis_index("c")
    s = lax.axis_index("s")
    pltpu.sync_copy(ones_h, ones_v)

    rows_per_core = EROWS_P // NC
    rows_per_tile = rows_per_core // NS
    nchunks = rows_per_tile // CH
    for r, ih in enumerate((eps_h, epd_h, ens_h, end_h, pcd_h)):
      _striped(lambda o, n: pltpu.sync_copy(z16_h.at[pl.ds(o, n)],
                                            acc.at[pl.ds(o, n)]), s)
      plsc.subcore_barrier()

      @pl.loop(0, nchunks)
      def _(g, ih=ih):
        row0 = c * rows_per_core + s * rows_per_tile + g * CH
        pltpu.sync_copy(ih.at[pl.ds(row0, CH)], idx_v)
        descs = [
            pltpu.async_copy(ones_v, acc.at[idx_v.at[b]], sem, add=True)
            for b in range(CH)
        ]
        for d_ in descs:
          d_.wait()

      plsc.subcore_barrier()

      def wout(o, n, r=r):
        pltpu.sync_copy(acc.at[pl.ds(o, n)], out_h.at[c, r, pl.ds(o, n)])

      _striped(wout, s)

  return k(eps, epd, ens, end_, pcd, z16, ones16)


def _sc_dual_scatter(hp, hn, eps, epd, ens, end_, z16):
  @functools.partial(
      pl.kernel,
      mesh=_MESH,
      compiler_params=pltpu.CompilerParams(use_tc_tiling_on_sc=False),
      out_type=jax.ShapeDtypeStruct((NC, N_JOB, D), f32),
      scratch_types=[
          pltpu.VMEM((CHD, LANES), jnp.int32),
          pltpu.VMEM((CHD, LANES), jnp.int32),
          pltpu.VMEM((CHD, LANES, D), f32),
          pltpu.VMEM_SHARED((N_JOB + 1, D), f32),
          pltpu.SemaphoreType.DMA,
          pltpu.SemaphoreType.DMA,
      ],
  )
  def k(hp_h, hn_h, eps_h, epd_h, ens_h, end_h, z16_h, out_h,
        src_v, dst_v, rows_v, acc, gsem, ssem):
    c = lax.axis_index("c")
    s = lax.axis_index("s")
    _striped(lambda o, n: pltpu.sync_copy(z16_h.at[pl.ds(o, n)],
                                          acc.at[pl.ds(o, n)]), s)
    plsc.subcore_barrier()

    rows_per_tile = EROWS_P // NS
    nchunks = rows_per_tile // CHD

    def run(table_h, src_h, dst_h):
      @pl.loop(0, nchunks)
      def _(g):
        row0 = s * rows_per_tile + g * CHD
        pltpu.sync_copy(src_h.at[pl.ds(row0, CHD)], src_v)
        pltpu.sync_copy(dst_h.at[pl.ds(row0, CHD)], dst_v)
        gd = [
            pltpu.async_copy(table_h.at[src_v.at[b]], rows_v.at[b], gsem)
            for b in range(CHD)
        ]
        for d_ in gd:
          d_.wait()
        sd = [
            pltpu.async_copy(rows_v.at[b], acc.at[dst_v.at[b]], ssem,
                             add=True)
            for b in range(CHD)
        ]
        for d_ in sd:
          d_.wait()

    @pl.when(c == 0)
    def _():
      run(hp_h, eps_h, epd_h)

    @pl.when(c == 1)
    def _():
      run(hn_h, ens_h, end_h)

    plsc.subcore_barrier()

    def wout(o, n):
      pltpu.sync_copy(acc.at[pl.ds(o, n)], out_h.at[c, pl.ds(o, n)])

    _striped(wout, s)

  return k(hp, hn, eps, epd, ens, end_, z16)


def _sc_sage(hw, pcs, pcd, z16):
  @functools.partial(
      pl.kernel,
      mesh=_MESH,
      compiler_params=pltpu.CompilerParams(use_tc_tiling_on_sc=False),
      out_type=jax.ShapeDtypeStruct((NC, N_JOB, D), f32),
      scratch_types=[
          pltpu.VMEM((CH, LANES), jnp.int32),
          pltpu.VMEM((CH, LANES), jnp.int32),
          pltpu.VMEM((CH, LANES, D), f32),
          pltpu.VMEM_SHARED((N_JOB + 1, D), f32),
          pltpu.SemaphoreType.DMA,
          pltpu.SemaphoreType.DMA,
      ],
  )
  def k(hw_h, src_h, dst_h, z16_h, out_h, src_v, dst_v, rows_v, acc,
        gsem, ssem):
    c = lax.axis_index("c")
    s = lax.axis_index("s")
    _striped(lambda o, n: pltpu.sync_copy(z16_h.at[pl.ds(o, n)],
                                          acc.at[pl.ds(o, n)]), s)
    plsc.subcore_barrier()

    rows_per_core = EROWS_P // NC
    rows_per_tile = rows_per_core // NS
    nchunks = rows_per_tile // CH

    @pl.loop(0, nchunks)
    def _(g):
      row0 = c * rows_per_core + s * rows_per_tile + g * CH
      pltpu.sync_copy(src_h.at[pl.ds(row0, CH)], src_v)
      pltpu.sync_copy(dst_h.at[pl.ds(row0, CH)], dst_v)
      gd = [
          pltpu.async_copy(hw_h.at[src_v.at[b]], rows_v.at[b], gsem)
          for b in range(CH)
      ]
      for d_ in gd:
        d_.wait()
      sd = [
          pltpu.async_copy(rows_v.at[b], acc.at[dst_v.at[b]], ssem, add=True)
          for b in range(CH)
      ]
      for d_ in sd:
        d_.wait()

    plsc.subcore_barrier()

    def wout(o, n):
      pltpu.sync_copy(acc.at[pl.ds(o, n)], out_h.at[c, pl.ds(o, n)])

    _striped(wout, s)

  return k(hw, pcs, pcd, z16)


def _sc_final_gather(hw, h2, pcs, pcdg):
  @functools.partial(
      pl.kernel,
      mesh=_MESH,
      compiler_params=pltpu.CompilerParams(use_tc_tiling_on_sc=False),
      out_type=[
          jax.ShapeDtypeStruct((EROWS_P, LANES, D), f32),
          jax.ShapeDtypeStruct((EROWS_P, LANES, D), f32),
      ],
      scratch_types=[
          pltpu.VMEM((CH, LANES), jnp.int32),
          pltpu.VMEM((CH, LANES), jnp.int32),
          pltpu.VMEM((CH, LANES, D), f32),
          pltpu.VMEM((CH, LANES, D), f32),
          pltpu.SemaphoreType.DMA,
          pltpu.SemaphoreType.DMA,
      ],
  )
  def k(hw_h, h2_h, src_h, dst_h, a_h, b_h, src_v, dst_v, arows_v, brows_v,
        gsem, ssem):
    c = lax.axis_index("c")
    s = lax.axis_index("s")
    rows_per_core = EROWS_P // NC
    rows_per_tile = rows_per_core // NS
    nchunks = rows_per_tile // CH

    @pl.loop(0, nchunks)
    def _(g):
      row0 = c * rows_per_core + s * rows_per_tile + g * CH
      pltpu.sync_copy(src_h.at[pl.ds(row0, CH)], src_v)
      pltpu.sync_copy(dst_h.at[pl.ds(row0, CH)], dst_v)
      gd = [
          pltpu.async_copy(hw_h.at[src_v.at[b]], arows_v.at[b], gsem)
          for b in range(CH)
      ]
      gd += [
          pltpu.async_copy(h2_h.at[dst_v.at[b]], brows_v.at[b], gsem)
          for b in range(CH)
      ]
      for d_ in gd:
        d_.wait()
      sd = [
          pltpu.async_copy(arows_v, a_h.at[pl.ds(row0, CH)], ssem),
          pltpu.async_copy(brows_v, b_h.at[pl.ds(row0, CH)], ssem),
      ]
      for d_ in sd:
        d_.wait()

  return k(hw, h2, pcs, pcdg)


_BN = 800


def _tc_embed(x, w, b):
  n, din = x.shape
  bn = 2000
  grid = n // bn

  def body(x_r, w_r, b_r, o_r):
    o_r[...] = jnp.dot(x_r[...], w_r[...],
                       preferred_element_type=f32) + b_r[...]

  return pl.pallas_call(
      body,
      grid=(grid,),
      in_specs=[
          pl.BlockSpec((bn, din), lambda i: (i, 0)),
          pl.BlockSpec((din, D), lambda i: (0, 0)),
          pl.BlockSpec((1, D), lambda i: (0, 0)),
      ],
      out_specs=pl.BlockSpec((bn, D), lambda i: (i, 0)),
      out_shape=jax.ShapeDtypeStruct((n, D), f32),
  )(x, w, b.reshape(1, D))


def _tc_prescale(counts, h0):
  def body(c_r, h_r, hp_r, hn_r, dop_r, don_r, dip_r, din_r, ivp_r):
    ctot = c_r[0, :, :, 0:1] + c_r[1, :, :, 0:1]
    dop = lax.rsqrt(jnp.maximum(ctot[0], 1.0))
    dip = lax.rsqrt(jnp.maximum(ctot[1], 1.0))
    don = lax.rsqrt(jnp.maximum(ctot[2], 1.0))
    din = lax.rsqrt(jnp.maximum(ctot[3], 1.0))
    ivp = 1.0 / jnp.maximum(ctot[4], 1.0)
    h = h_r[...]
    hp_r[...] = h * dop
    hn_r[...] = h * don
    dop_r[...] = dop
    don_r[...] = don
    dip_r[...] = dip
    din_r[...] = din
    ivp_r[...] = ivp

  grid = N_JOB // _BN
  vspec = pl.BlockSpec((_BN, 1), lambda i: (i, 0))
  hspec = pl.BlockSpec((_BN, D), lambda i: (i, 0))
  return pl.pallas_call(
      body,
      grid=(grid,),
      in_specs=[
          pl.BlockSpec((2, 5, _BN, D), lambda i: (0, 0, i, 0)),
          hspec,
      ],
      out_specs=[hspec, hspec, vspec, vspec, vspec, vspec, vspec],
      out_shape=[
          jax.ShapeDtypeStruct((N_JOB, D), f32),
          jax.ShapeDtypeStruct((N_JOB, D), f32),
      ] + [jax.ShapeDtypeStruct((N_JOB, 1), f32)] * 5,
  )(counts, h0)


def _tc_sage_c(s2, ivp, w_neigh, b_sage):
  def body(s_r, ivp_r, w_r, b_r, o_r):
    mean = (s_r[0] + s_r[1]) * ivp_r[...]
    o_r[...] = jnp.dot(mean, w_r[...], preferred_element_type=f32) + b_r[...]

  grid = N_JOB // _BN
  return pl.pallas_call(
      body,
      grid=(grid,),
      in_specs=[
          pl.BlockSpec((2, _BN, D), lambda i: (0, i, 0)),
          pl.BlockSpec((_BN, 1), lambda i: (i, 0)),
          pl.BlockSpec((D, D), lambda i: (0, 0)),
          pl.BlockSpec((1, D), lambda i: (0, 0)),
      ],
      out_specs=pl.BlockSpec((_BN, D), lambda i: (i, 0)),
      out_shape=jax.ShapeDtypeStruct((N_JOB, D), f32),
  )(s2, ivp, w_neigh, b_sage.reshape(1, D))


def _tc_combine(s_pn, h, c_term, dip, din, dop, don, w_prec, w_next, w_self,
                bpn, last):
  def body(s_r, h_r, c_r, dip_r, din_r, dop_r, don_r, wp_r, wn_r, ws_r,
           bpn_r, *outs):
    hp = jnp.dot(s_r[0], wp_r[...], preferred_element_type=f32) * dip_r[...]
    hn = jnp.dot(s_r[1], wn_r[...], preferred_element_type=f32) * din_r[...]
    hs = jnp.dot(h_r[...], ws_r[...], preferred_element_type=f32)
    hnew = hp + hn + hs + c_r[...] + bpn_r[...]
    outs[0][...] = hnew
    if not last:
      outs[1][...] = hnew * dop_r[...]
      outs[2][...] = hnew * don_r[...]

  grid = N_JOB // _BN
  hspec = pl.BlockSpec((_BN, D), lambda i: (i, 0))
  vspec = pl.BlockSpec((_BN, 1), lambda i: (i, 0))
  wspec = pl.BlockSpec((D, D), lambda i: (0, 0))
  n_out = 1 if last else 3
  return pl.pallas_call(
      body,
      grid=(grid,),
      in_specs=[
          pl.BlockSpec((2, _BN, D), lambda i: (0, i, 0)),
          hspec, hspec, vspec, vspec, vspec, vspec, wspec, wspec, wspec,
          pl.BlockSpec((1, D), lambda i: (0, 0)),
      ],
      out_specs=[hspec] * n_out,
      out_shape=[jax.ShapeDtypeStruct((N_JOB, D), f32)] * n_out,
  )(s_pn, h, c_term, dip, din, dop, don, w_prec, w_next, w_self, bpn)


def _tc_dot(a, b):
  bn = 2048
  grid = (E + bn - 1) // bn

  def body(a_r, b_r, o_r):
    o_r[...] = jnp.sum(a_r[...] * b_r[...], axis=1, keepdims=True)

  return pl.pallas_call(
      body,
      grid=(grid,),
      in_specs=[
          pl.BlockSpec((bn, D), lambda i: (i, 0)),
          pl.BlockSpec((bn, D), lambda i: (i, 0)),
      ],
      out_specs=pl.BlockSpec((bn, 1), lambda i: (i, 0)),
      out_shape=jax.ShapeDtypeStruct((E, 1), f32),
  )(a, b)


def _pad2d(a, padval):
  pad = jnp.full((EPAD,), padval, jnp.int32)
  return jnp.concatenate([a, pad]).reshape(EROWS_P, LANES)


def kernel(x_job, x_worker, edge_precede, edge_next, proc_src, proc_dst,
           W_emb_job, b_emb_job, W_emb_worker, b_emb_worker,
           W_prec, b_prec, W_next, b_next, W_self, W_neigh, b_sage):
  DUMMY = N_JOB

  eps_g = _pad2d(edge_precede[0], 0)
  eps_s = _pad2d(edge_precede[0], DUMMY)
  epd_s = _pad2d(edge_precede[1], DUMMY)
  ens_g = _pad2d(edge_next[0], 0)
  ens_s = _pad2d(edge_next[0], DUMMY)
  end_s = _pad2d(edge_next[1], DUMMY)
  pcs_g = _pad2d(proc_src, 0)
  pcd_s = _pad2d(proc_dst, DUMMY)
  pcd_g = _pad2d(proc_dst, 0)

  z16 = jnp.zeros((N_JOB, D), f32)
  ones16 = jnp.ones((LANES, D), f32)
  bpn = (b_prec + b_next).reshape(1, D)

  h0 = _tc_embed(x_job, W_emb_job, b_emb_job)
  hw = _tc_embed(x_worker, W_emb_worker, b_emb_worker)

  counts = _sc_degrees(eps_s, epd_s, ens_s, end_s, pcd_s, z16, ones16)

  hp, hn, dop, don, dip, din, ivp = _tc_prescale(counts, h0)

  hw_d, counts = lax.optimization_barrier((hw, counts))
  s2 = _sc_sage(hw_d, pcs_g, pcd_s, z16)
  c_term = _tc_sage_c(s2, ivp, W_neigh, b_sage)
  hp, s2 = lax.optimization_barrier((hp, s2))

  h = h0
  for loop in range(NUM_LOOPS):
    last = loop == NUM_LOOPS - 1
    s_pn = _sc_dual_scatter(hp, hn, eps_g, epd_s, ens_g, end_s, z16)
    outs = _tc_combine(s_pn, h, c_term, dip, din, dop, don,
                       W_prec, W_next, W_self, bpn, last)
    if last:
      h = outs[0]
    else:
      h, hp, hn = outs

  a_rows, b_rows = _sc_final_gather(hw, h, pcs_g, pcd_g)
  a_rows = a_rows.reshape(EROWS_P * LANES, D)
  b_rows = b_rows.reshape(EROWS_P * LANES, D)
  return _tc_dot(a_rows, b_rows)


# speedup vs baseline: 14.5640x; 2.4554x over previous
import functools

import jax
import jax.numpy as jnp
from jax import lax
from jax.experimental import pallas as pl
from jax.experimental.pallas import tpu as pltpu
from jax.experimental.pallas import tpu_sc as plsc

N_JOB = 100000
N_WORKER = 10000
E = 1600000
D = 16
NUM_LOOPS = 2

NC = 2
NS = 16
LANES = 128
NP = 100352
PR = NP // 8
NWP = 10240
PRW = NWP // 8
EROWS_P = 12544
EPAD = EROWS_P * LANES - E
EP = EROWS_P * LANES
PRE = EP // 8
CH = 8
ST = NP // NS

_MESH = plsc.VectorSubcoreMesh(core_axis_name="c", subcore_axis_name="s")

f32 = jnp.float32


def _sc_degrees(eps, epd, ens, end_, pcd, z16, ones16):
  @functools.partial(
      pl.kernel,
      mesh=_MESH,
      compiler_params=pltpu.CompilerParams(use_tc_tiling_on_sc=False),
      out_type=jax.ShapeDtypeStruct((NC, 5, NP, D), f32),
      scratch_types=[
          pltpu.VMEM((CH, LANES), jnp.int32),
          pltpu.VMEM((LANES, D), f32),
          pltpu.VMEM_SHARED((NP + 1, D), f32),
          pltpu.SemaphoreType.DMA,
      ],
  )
  def k(eps_h, epd_h, ens_h, end_h, pcd_h, z16_h, ones_h, out_h,
        idx_v, ones_v, acc, sem):
    c = lax.axis_index("c")
    s = lax.axis_index("s")
    pltpu.sync_copy(ones_h, ones_v)

    rows_per_core = EROWS_P // NC
    rows_per_tile = rows_per_core // NS
    nchunks = rows_per_tile // CH
    for r, ih in enumerate((eps_h, epd_h, ens_h, end_h, pcd_h)):
      pltpu.sync_copy(z16_h.at[pl.ds(s * ST, ST)], acc.at[pl.ds(s * ST, ST)])
      plsc.subcore_barrier()

      @pl.loop(0, nchunks)
      def _(g, ih=ih):
        row0 = c * rows_per_core + s * rows_per_tile + g * CH
        pltpu.sync_copy(ih.at[pl.ds(row0, CH)], idx_v)
        descs = [
            pltpu.async_copy(ones_v, acc.at[idx_v.at[b]], sem, add=True)
            for b in range(CH)
        ]
        for d_ in descs:
          d_.wait()

      plsc.subcore_barrier()
      pltpu.sync_copy(acc.at[pl.ds(s * ST, ST)],
                      out_h.at[c, r, pl.ds(s * ST, ST)])

  return k(eps, epd, ens, end_, pcd, z16, ones16)


def _sc_dual_scatter(hp, hn, eps, epd, ens, end_, z16):
  @functools.partial(
      pl.kernel,
      mesh=_MESH,
      compiler_params=pltpu.CompilerParams(use_tc_tiling_on_sc=False),
      out_type=jax.ShapeDtypeStruct((NC, NP, D), f32),
      scratch_types=[
          pltpu.VMEM((CH, LANES), jnp.int32),
          pltpu.VMEM((CH, LANES), jnp.int32),
          pltpu.VMEM((CH, LANES, D), f32),
          pltpu.VMEM_SHARED((NP + 1, D), f32),
          pltpu.SemaphoreType.DMA,
          pltpu.SemaphoreType.DMA,
      ],
  )
  def k(hp_h, hn_h, eps_h, epd_h, ens_h, end_h, z16_h, out_h,
        src_v, dst_v, rows_v, acc, gsem, ssem):
    c = lax.axis_index("c")
    s = lax.axis_index("s")
    pltpu.sync_copy(z16_h.at[pl.ds(s * ST, ST)], acc.at[pl.ds(s * ST, ST)])
    plsc.subcore_barrier()

    rows_per_tile = EROWS_P // NS
    nchunks = rows_per_tile // CH

    def run(table_h, src_h, dst_h):
      @pl.loop(0, nchunks)
      def _(g):
        row0 = s * rows_per_tile + g * CH
        pltpu.sync_copy(src_h.at[pl.ds(row0, CH)], src_v)
        pltpu.sync_copy(dst_h.at[pl.ds(row0, CH)], dst_v)
        gd = [
            pltpu.async_copy(table_h.at[src_v.at[b]], rows_v.at[b], gsem)
            for b in range(CH)
        ]
        for d_ in gd:
          d_.wait()
        sd = [
            pltpu.async_copy(rows_v.at[b], acc.at[dst_v.at[b]], ssem,
                             add=True)
            for b in range(CH)
        ]
        for d_ in sd:
          d_.wait()

    @pl.when(c == 0)
    def _():
      run(hp_h, eps_h, epd_h)

    @pl.when(c == 1)
    def _():
      run(hn_h, ens_h, end_h)

    plsc.subcore_barrier()
    pltpu.sync_copy(acc.at[pl.ds(s * ST, ST)],
                    out_h.at[c, pl.ds(s * ST, ST)])

  return k(hp, hn, eps, epd, ens, end_, z16)


def _sc_sage(hw, pcs, pcd, z16):
  @functools.partial(
      pl.kernel,
      mesh=_MESH,
      compiler_params=pltpu.CompilerParams(use_tc_tiling_on_sc=False),
      out_type=jax.ShapeDtypeStruct((NC, NP, D), f32),
      scratch_types=[
          pltpu.VMEM((CH, LANES), jnp.int32),
          pltpu.VMEM((CH, LANES), jnp.int32),
          pltpu.VMEM((CH, LANES, D), f32),
          pltpu.VMEM_SHARED((NP + 1, D), f32),
          pltpu.SemaphoreType.DMA,
          pltpu.SemaphoreType.DMA,
      ],
  )
  def k(hw_h, src_h, dst_h, z16_h, out_h, src_v, dst_v, rows_v, acc,
        gsem, ssem):
    c = lax.axis_index("c")
    s = lax.axis_index("s")
    pltpu.sync_copy(z16_h.at[pl.ds(s * ST, ST)], acc.at[pl.ds(s * ST, ST)])
    plsc.subcore_barrier()

    rows_per_core = EROWS_P // NC
    rows_per_tile = rows_per_core // NS
    nchunks = rows_per_tile // CH

    @pl.loop(0, nchunks)
    def _(g):
      row0 = c * rows_per_core + s * rows_per_tile + g * CH
      pltpu.sync_copy(src_h.at[pl.ds(row0, CH)], src_v)
      pltpu.sync_copy(dst_h.at[pl.ds(row0, CH)], dst_v)
      gd = [
          pltpu.async_copy(hw_h.at[src_v.at[b]], rows_v.at[b], gsem)
          for b in range(CH)
      ]
      for d_ in gd:
        d_.wait()
      sd = [
          pltpu.async_copy(rows_v.at[b], acc.at[dst_v.at[b]], ssem, add=True)
          for b in range(CH)
      ]
      for d_ in sd:
        d_.wait()

    plsc.subcore_barrier()
    pltpu.sync_copy(acc.at[pl.ds(s * ST, ST)],
                    out_h.at[c, pl.ds(s * ST, ST)])

  return k(hw, pcs, pcd, z16)


def _sc_final_gather(hw, h2, pcs, pcdg):
  @functools.partial(
      pl.kernel,
      mesh=_MESH,
      compiler_params=pltpu.CompilerParams(use_tc_tiling_on_sc=False),
      out_type=[
          jax.ShapeDtypeStruct((EROWS_P, LANES, D), f32),
          jax.ShapeDtypeStruct((EROWS_P, LANES, D), f32),
      ],
      scratch_types=[
          pltpu.VMEM((CH, LANES), jnp.int32),
          pltpu.VMEM((CH, LANES), jnp.int32),
          pltpu.VMEM((CH, LANES, D), f32),
          pltpu.VMEM((CH, LANES, D), f32),
          pltpu.SemaphoreType.DMA,
          pltpu.SemaphoreType.DMA,
      ],
  )
  def k(hw_h, h2_h, src_h, dst_h, a_h, b_h, src_v, dst_v, arows_v, brows_v,
        gsem, ssem):
    c = lax.axis_index("c")
    s = lax.axis_index("s")
    rows_per_core = EROWS_P // NC
    rows_per_tile = rows_per_core // NS
    nchunks = rows_per_tile // CH

    @pl.loop(0, nchunks)
    def _(g):
      row0 = c * rows_per_core + s * rows_per_tile + g * CH
      pltpu.sync_copy(src_h.at[pl.ds(row0, CH)], src_v)
      pltpu.sync_copy(dst_h.at[pl.ds(row0, CH)], dst_v)
      gd = [
          pltpu.async_copy(hw_h.at[src_v.at[b]], arows_v.at[b], gsem)
          for b in range(CH)
      ]
      gd += [
          pltpu.async_copy(h2_h.at[dst_v.at[b]], brows_v.at[b], gsem)
          for b in range(CH)
      ]
      for d_ in gd:
        d_.wait()
      sd = [
          pltpu.async_copy(arows_v, a_h.at[pl.ds(row0, CH)], ssem),
          pltpu.async_copy(brows_v, b_h.at[pl.ds(row0, CH)], ssem),
      ]
      for d_ in sd:
        d_.wait()

  return k(hw, h2, pcs, pcdg)


_BN = 784


def _tc_embed(x8, w8, b8):
  rows, din8 = x8.shape
  bn = rows // 16 if rows % 16 == 0 else rows
  grid = rows // bn

  def body(x_r, w_r, b_r, o_r):
    o_r[...] = jnp.dot(x_r[...], w_r[...],
                       preferred_element_type=f32) + b_r[...]

  return pl.pallas_call(
      body,
      grid=(grid,),
      in_specs=[
          pl.BlockSpec((bn, din8), lambda i: (i, 0)),
          pl.BlockSpec((din8, LANES), lambda i: (0, 0)),
          pl.BlockSpec((1, LANES), lambda i: (0, 0)),
      ],
      out_specs=pl.BlockSpec((bn, LANES), lambda i: (i, 0)),
      out_shape=jax.ShapeDtypeStruct((rows, LANES), f32),
  )(x8, w8, b8)


def _tc_prescale(counts_p, h0p):
  def body(c_r, h_r, hp_r, hn_r, dop_r, don_r, dip_r, din_r, ivp_r):
    ctot = c_r[0] + c_r[1]
    dop = lax.rsqrt(jnp.maximum(ctot[0], 1.0))
    dip = lax.rsqrt(jnp.maximum(ctot[1], 1.0))
    don = lax.rsqrt(jnp.maximum(ctot[2], 1.0))
    din = lax.rsqrt(jnp.maximum(ctot[3], 1.0))
    ivp = 1.0 / jnp.maximum(ctot[4], 1.0)
    h = h_r[...]
    hp_r[...] = h * dop
    hn_r[...] = h * don
    dop_r[...] = dop
    don_r[...] = don
    dip_r[...] = dip
    din_r[...] = din
    ivp_r[...] = ivp

  grid = PR // _BN
  pspec = pl.BlockSpec((_BN, LANES), lambda i: (i, 0))
  return pl.pallas_call(
      body,
      grid=(grid,),
      in_specs=[
          pl.BlockSpec((2, 5, _BN, LANES), lambda i: (0, 0, i, 0)),
          pspec,
      ],
      out_specs=[pspec] * 7,
      out_shape=[jax.ShapeDtypeStruct((PR, LANES), f32)] * 7,
  )(counts_p, h0p)


def _tc_sage_c(s2p, ivp, w_neigh8, b_sage8):
  def body(s_r, ivp_r, w_r, b_r, o_r):
    mean = (s_r[0] + s_r[1]) * ivp_r[...]
    o_r[...] = jnp.dot(mean, w_r[...], preferred_element_type=f32) + b_r[...]

  grid = PR // _BN
  pspec = pl.BlockSpec((_BN, LANES), lambda i: (i, 0))
  return pl.pallas_call(
      body,
      grid=(grid,),
      in_specs=[
          pl.BlockSpec((2, _BN, LANES), lambda i: (0, i, 0)),
          pspec,
          pl.BlockSpec((LANES, LANES), lambda i: (0, 0)),
          pl.BlockSpec((1, LANES), lambda i: (0, 0)),
      ],
      out_specs=pspec,
      out_shape=jax.ShapeDtypeStruct((PR, LANES), f32),
  )(s2p, ivp, w_neigh8, b_sage8)


def _tc_combine(s_pn, h, c_term, dip, din, dop, don, wp8, wn8, ws8,
                bpn8, last):
  def body(s_r, h_r, c_r, dip_r, din_r, dop_r, don_r, wp_r, wn_r, ws_r,
           bpn_r, *outs):
    hp = jnp.dot(s_r[0], wp_r[...], preferred_element_type=f32) * dip_r[...]
    hn = jnp.dot(s_r[1], wn_r[...], preferred_element_type=f32) * din_r[...]
    hs = jnp.dot(h_r[...], ws_r[...], preferred_element_type=f32)
    hnew = hp + hn + hs + c_r[...] + bpn_r[...]
    outs[0][...] = hnew
    if not last:
      outs[1][...] = hnew * dop_r[...]
      outs[2][...] = hnew * don_r[...]

  grid = PR // _BN
  pspec = pl.BlockSpec((_BN, LANES), lambda i: (i, 0))
  wspec = pl.BlockSpec((LANES, LANES), lambda i: (0, 0))
  n_out = 1 if last else 3
  return pl.pallas_call(
      body,
      grid=(grid,),
      in_specs=[
          pl.BlockSpec((2, _BN, LANES), lambda i: (0, i, 0)),
          pspec, pspec, pspec, pspec, pspec, pspec, wspec, wspec, wspec,
          pl.BlockSpec((1, LANES), lambda i: (0, 0)),
      ],
      out_specs=[pspec] * n_out,
      out_shape=[jax.ShapeDtypeStruct((PR, LANES), f32)] * n_out,
  )(s_pn, h, c_term, dip, din, dop, don, wp8, wn8, ws8, bpn8)


def _tc_dot(ap, bp, k8t):
  bn = 2048
  grid = PRE // bn

  def body(a_r, b_r, k_r, o_r):
    p = a_r[...] * b_r[...]
    o_r[...] = lax.dot_general(k_r[...], p, (((1,), (1,)), ((), ())),
                               preferred_element_type=f32)

  return pl.pallas_call(
      body,
      grid=(grid,),
      in_specs=[
          pl.BlockSpec((bn, LANES), lambda i: (i, 0)),
          pl.BlockSpec((bn, LANES), lambda i: (i, 0)),
          pl.BlockSpec((8, LANES), lambda i: (0, 0)),
      ],
      out_specs=pl.BlockSpec((8, bn), lambda i: (0, i)),
      out_shape=jax.ShapeDtypeStruct((8, PRE), f32),
  )(ap, bp, k8t)


def _pad2d(a, padval):
  pad = jnp.full((EPAD,), padval, jnp.int32)
  return jnp.concatenate([a, pad]).reshape(EROWS_P, LANES)


def _kron8(w):
  return jnp.kron(jnp.eye(8, dtype=f32), w)


def kernel(x_job, x_worker, edge_precede, edge_next, proc_src, proc_dst,
           W_emb_job, b_emb_job, W_emb_worker, b_emb_worker,
           W_prec, b_prec, W_next, b_next, W_self, W_neigh, b_sage):
  DUMMY = NP

  eps_g = _pad2d(edge_precede[0], 0)
  eps_s = _pad2d(edge_precede[0], DUMMY)
  epd_s = _pad2d(edge_precede[1], DUMMY)
  ens_g = _pad2d(edge_next[0], 0)
  ens_s = _pad2d(edge_next[0], DUMMY)
  end_s = _pad2d(edge_next[1], DUMMY)
  pcs_g = _pad2d(proc_src, 0)
  pcd_s = _pad2d(proc_dst, DUMMY)
  pcd_g = _pad2d(proc_dst, 0)

  x8 = jnp.concatenate(
      [x_job.reshape(N_JOB // 8, 56),
       jnp.zeros(((NP - N_JOB) // 8, 56), f32)])
  xw8 = jnp.concatenate(
      [x_worker.reshape(N_WORKER // 8, 24),
       jnp.zeros(((NWP - N_WORKER) // 8, 24), f32)])
  z16 = jnp.zeros((NP, D), f32)
  ones16 = jnp.ones((LANES, D), f32)
  wej8 = _kron8(W_emb_job)
  wew8 = _kron8(W_emb_worker)
  wp8 = _kron8(W_prec)
  wn8 = _kron8(W_next)
  ws8 = _kron8(W_self)
  wng8 = _kron8(W_neigh)
  bej8 = jnp.tile(b_emb_job, 8).reshape(1, LANES)
  bew8 = jnp.tile(b_emb_worker, 8).reshape(1, LANES)
  bpn8 = jnp.tile(b_prec + b_next, 8).reshape(1, LANES)
  bs8 = jnp.tile(b_sage, 8).reshape(1, LANES)
  k8t = jnp.kron(jnp.eye(8, dtype=f32), jnp.ones((1, 16), f32))

  h0p = _tc_embed(x8, wej8, bej8)
  hwp = _tc_embed(xw8, wew8, bew8)

  counts = _sc_degrees(eps_s, epd_s, ens_s, end_s, pcd_s, z16, ones16)
  counts_p = counts.reshape(NC, 5, PR, LANES)

  hp, hn, dop, don, dip, din, ivp = _tc_prescale(counts_p, h0p)

  hwp, counts_p = lax.optimization_barrier((hwp, counts_p))
  s2 = _sc_sage(hwp.reshape(NWP, D), pcs_g, pcd_s, z16)
  c_term = _tc_sage_c(s2.reshape(NC, PR, LANES), ivp, wng8, bs8)
  hp, s2 = lax.optimization_barrier((hp, s2))

  h = h0p
  for loop in range(NUM_LOOPS):
    last = loop == NUM_LOOPS - 1
    s_pn = _sc_dual_scatter(hp.reshape(NP, D), hn.reshape(NP, D),
                            eps_g, epd_s, ens_g, end_s, z16)
    outs = _tc_combine(s_pn.reshape(NC, PR, LANES), h, c_term,
                       dip, din, dop, don, wp8, wn8, ws8, bpn8, last)
    if last:
      h = outs[0]
    else:
      h, hp, hn = outs

  a_rows, b_rows = _sc_final_gather(hwp.reshape(NWP, D), h.reshape(NP, D),
                                    pcs_g, pcd_g)
  scores_t = _tc_dot(a_rows.reshape(PRE, LANES), b_rows.reshape(PRE, LANES),
                     k8t)
  return scores_t.T.reshape(EP, 1)[:E]


# speedup vs baseline: 15.4088x; 1.0580x over previous
import functools

import jax
import jax.numpy as jnp
from jax import lax
from jax.experimental import pallas as pl
from jax.experimental.pallas import tpu as pltpu
from jax.experimental.pallas import tpu_sc as plsc

N_JOB = 100000
N_WORKER = 10000
E = 1600000
D = 16
NUM_LOOPS = 2

NC = 2
NS = 16
LANES = 128
NP = 100352
PR = NP // 8
NWP = 10240
PRW = NWP // 8
EROWS_P = 12544
EPAD = EROWS_P * LANES - E
EP = EROWS_P * LANES
PRE = EP // 8
CH = 8
ST = NP // NS

_MESH = plsc.VectorSubcoreMesh(core_axis_name="c", subcore_axis_name="s")

f32 = jnp.float32


def _sc_degrees(eps, epd, ens, end_, pcd, z16, ones16):
  @functools.partial(
      pl.kernel,
      mesh=_MESH,
      compiler_params=pltpu.CompilerParams(use_tc_tiling_on_sc=False),
      out_type=jax.ShapeDtypeStruct((NC, 5, NP, D), f32),
      scratch_types=[
          pltpu.VMEM((CH, LANES), jnp.int32),
          pltpu.VMEM((LANES, D), f32),
          pltpu.VMEM_SHARED((NP + 1, D), f32),
          pltpu.SemaphoreType.DMA,
      ],
  )
  def k(eps_h, epd_h, ens_h, end_h, pcd_h, z16_h, ones_h, out_h,
        idx_v, ones_v, acc, sem):
    c = lax.axis_index("c")
    s = lax.axis_index("s")
    pltpu.sync_copy(ones_h, ones_v)

    rows_per_core = EROWS_P // NC
    rows_per_tile = rows_per_core // NS
    nchunks = rows_per_tile // CH
    for r, ih in enumerate((eps_h, epd_h, ens_h, end_h, pcd_h)):
      pltpu.sync_copy(z16_h.at[pl.ds(s * ST, ST)], acc.at[pl.ds(s * ST, ST)])
      plsc.subcore_barrier()

      @pl.loop(0, nchunks)
      def _(g, ih=ih):
        row0 = c * rows_per_core + s * rows_per_tile + g * CH
        pltpu.sync_copy(ih.at[pl.ds(row0, CH)], idx_v)
        descs = [
            pltpu.async_copy(ones_v, acc.at[idx_v.at[b]], sem, add=True)
            for b in range(CH)
        ]
        for d_ in descs:
          d_.wait()

      plsc.subcore_barrier()
      pltpu.sync_copy(acc.at[pl.ds(s * ST, ST)],
                      out_h.at[c, r, pl.ds(s * ST, ST)])

  return k(eps, epd, ens, end_, pcd, z16, ones16)


def _sc_dual_scatter(hp, hn, eps, epd, ens, end_, z16):
  @functools.partial(
      pl.kernel,
      mesh=_MESH,
      compiler_params=pltpu.CompilerParams(use_tc_tiling_on_sc=False),
      out_type=jax.ShapeDtypeStruct((NC, NP, D), f32),
      scratch_types=[
          pltpu.VMEM((CH, LANES), jnp.int32),
          pltpu.VMEM((CH, LANES), jnp.int32),
          pltpu.VMEM((CH, LANES, D), f32),
          pltpu.VMEM_SHARED((NP + 1, D), f32),
          pltpu.SemaphoreType.DMA,
          pltpu.SemaphoreType.DMA,
      ],
  )
  def k(hp_h, hn_h, eps_h, epd_h, ens_h, end_h, z16_h, out_h,
        src_v, dst_v, rows_v, acc, gsem, ssem):
    c = lax.axis_index("c")
    s = lax.axis_index("s")
    pltpu.sync_copy(z16_h.at[pl.ds(s * ST, ST)], acc.at[pl.ds(s * ST, ST)])
    plsc.subcore_barrier()

    rows_per_tile = EROWS_P // NS
    nchunks = rows_per_tile // CH

    def run(table_h, src_h, dst_h):
      @pl.loop(0, nchunks)
      def _(g):
        row0 = s * rows_per_tile + g * CH
        pltpu.sync_copy(src_h.at[pl.ds(row0, CH)], src_v)
        pltpu.sync_copy(dst_h.at[pl.ds(row0, CH)], dst_v)
        gd = [
            pltpu.async_copy(table_h.at[src_v.at[b]], rows_v.at[b], gsem)
            for b in range(CH)
        ]
        sd = []
        for b in range(CH):
          gd[b].wait()
          sd.append(pltpu.async_copy(rows_v.at[b], acc.at[dst_v.at[b]],
                                     ssem, add=True))
        for d_ in sd:
          d_.wait()

    @pl.when(c == 0)
    def _():
      run(hp_h, eps_h, epd_h)

    @pl.when(c == 1)
    def _():
      run(hn_h, ens_h, end_h)

    plsc.subcore_barrier()
    pltpu.sync_copy(acc.at[pl.ds(s * ST, ST)],
                    out_h.at[c, pl.ds(s * ST, ST)])

  return k(hp, hn, eps, epd, ens, end_, z16)


def _sc_sage(hw, pcs, pcd, z16):
  @functools.partial(
      pl.kernel,
      mesh=_MESH,
      compiler_params=pltpu.CompilerParams(use_tc_tiling_on_sc=False),
      out_type=jax.ShapeDtypeStruct((NC, NP, D), f32),
      scratch_types=[
          pltpu.VMEM((CH, LANES), jnp.int32),
          pltpu.VMEM((CH, LANES), jnp.int32),
          pltpu.VMEM((CH, LANES, D), f32),
          pltpu.VMEM_SHARED((NP + 1, D), f32),
          pltpu.SemaphoreType.DMA,
          pltpu.SemaphoreType.DMA,
      ],
  )
  def k(hw_h, src_h, dst_h, z16_h, out_h, src_v, dst_v, rows_v, acc,
        gsem, ssem):
    c = lax.axis_index("c")
    s = lax.axis_index("s")
    pltpu.sync_copy(z16_h.at[pl.ds(s * ST, ST)], acc.at[pl.ds(s * ST, ST)])
    plsc.subcore_barrier()

    rows_per_core = EROWS_P // NC
    rows_per_tile = rows_per_core // NS
    nchunks = rows_per_tile // CH

    @pl.loop(0, nchunks)
    def _(g):
      row0 = c * rows_per_core + s * rows_per_tile + g * CH
      pltpu.sync_copy(src_h.at[pl.ds(row0, CH)], src_v)
      pltpu.sync_copy(dst_h.at[pl.ds(row0, CH)], dst_v)
      gd = [
          pltpu.async_copy(hw_h.at[src_v.at[b]], rows_v.at[b], gsem)
          for b in range(CH)
      ]
      sd = []
      for b in range(CH):
        gd[b].wait()
        sd.append(pltpu.async_copy(rows_v.at[b], acc.at[dst_v.at[b]],
                                   ssem, add=True))
      for d_ in sd:
        d_.wait()

    plsc.subcore_barrier()
    pltpu.sync_copy(acc.at[pl.ds(s * ST, ST)],
                    out_h.at[c, pl.ds(s * ST, ST)])

  return k(hw, pcs, pcd, z16)


def _sc_final_gather(hw, h2, pcs, pcdg):
  @functools.partial(
      pl.kernel,
      mesh=_MESH,
      compiler_params=pltpu.CompilerParams(use_tc_tiling_on_sc=False),
      out_type=[
          jax.ShapeDtypeStruct((EROWS_P, LANES, D), f32),
          jax.ShapeDtypeStruct((EROWS_P, LANES, D), f32),
      ],
      scratch_types=[
          pltpu.VMEM((CH, LANES), jnp.int32),
          pltpu.VMEM((CH, LANES), jnp.int32),
          pltpu.VMEM((CH, LANES, D), f32),
          pltpu.VMEM((CH, LANES, D), f32),
          pltpu.SemaphoreType.DMA,
          pltpu.SemaphoreType.DMA,
      ],
  )
  def k(hw_h, h2_h, src_h, dst_h, a_h, b_h, src_v, dst_v, arows_v, brows_v,
        gsem, ssem):
    c = lax.axis_index("c")
    s = lax.axis_index("s")
    rows_per_core = EROWS_P // NC
    rows_per_tile = rows_per_core // NS
    nchunks = rows_per_tile // CH

    @pl.loop(0, nchunks)
    def _(g):
      row0 = c * rows_per_core + s * rows_per_tile + g * CH
      pltpu.sync_copy(src_h.at[pl.ds(row0, CH)], src_v)
      pltpu.sync_copy(dst_h.at[pl.ds(row0, CH)], dst_v)
      ga = [
          pltpu.async_copy(hw_h.at[src_v.at[b]], arows_v.at[b], gsem)
          for b in range(CH)
      ]
      gb = [
          pltpu.async_copy(h2_h.at[dst_v.at[b]], brows_v.at[b], gsem)
          for b in range(CH)
      ]
      for d_ in ga:
        d_.wait()
      sa = pltpu.async_copy(arows_v, a_h.at[pl.ds(row0, CH)], ssem)
      for d_ in gb:
        d_.wait()
      sb = pltpu.async_copy(brows_v, b_h.at[pl.ds(row0, CH)], ssem)
      sa.wait()
      sb.wait()

  return k(hw, h2, pcs, pcdg)


_BN = 784


def _tc_embed(x8, w8, b8):
  rows, din8 = x8.shape
  bn = rows // 16 if rows % 16 == 0 else rows
  grid = rows // bn

  def body(x_r, w_r, b_r, o_r):
    o_r[...] = jnp.dot(x_r[...], w_r[...],
                       preferred_element_type=f32) + b_r[...]

  return pl.pallas_call(
      body,
      grid=(grid,),
      in_specs=[
          pl.BlockSpec((bn, din8), lambda i: (i, 0)),
          pl.BlockSpec((din8, LANES), lambda i: (0, 0)),
          pl.BlockSpec((1, LANES), lambda i: (0, 0)),
      ],
      out_specs=pl.BlockSpec((bn, LANES), lambda i: (i, 0)),
      out_shape=jax.ShapeDtypeStruct((rows, LANES), f32),
  )(x8, w8, b8)


def _tc_prescale(counts_p, h0p):
  def body(c_r, h_r, hp_r, hn_r, dop_r, don_r, dip_r, din_r, ivp_r):
    ctot = c_r[0] + c_r[1]
    dop = lax.rsqrt(jnp.maximum(ctot[0], 1.0))
    dip = lax.rsqrt(jnp.maximum(ctot[1], 1.0))
    don = lax.rsqrt(jnp.maximum(ctot[2], 1.0))
    din = lax.rsqrt(jnp.maximum(ctot[3], 1.0))
    ivp = 1.0 / jnp.maximum(ctot[4], 1.0)
    h = h_r[...]
    hp_r[...] = h * dop
    hn_r[...] = h * don
    dop_r[...] = dop
    don_r[...] = don
    dip_r[...] = dip
    din_r[...] = din
    ivp_r[...] = ivp

  grid = PR // _BN
  pspec = pl.BlockSpec((_BN, LANES), lambda i: (i, 0))
  return pl.pallas_call(
      body,
      grid=(grid,),
      in_specs=[
          pl.BlockSpec((2, 5, _BN, LANES), lambda i: (0, 0, i, 0)),
          pspec,
      ],
      out_specs=[pspec] * 7,
      out_shape=[jax.ShapeDtypeStruct((PR, LANES), f32)] * 7,
  )(counts_p, h0p)


def _tc_sage_c(s2p, ivp, w_neigh8, b_sage8):
  def body(s_r, ivp_r, w_r, b_r, o_r):
    mean = (s_r[0] + s_r[1]) * ivp_r[...]
    o_r[...] = jnp.dot(mean, w_r[...], preferred_element_type=f32) + b_r[...]

  grid = PR // _BN
  pspec = pl.BlockSpec((_BN, LANES), lambda i: (i, 0))
  return pl.pallas_call(
      body,
      grid=(grid,),
      in_specs=[
          pl.BlockSpec((2, _BN, LANES), lambda i: (0, i, 0)),
          pspec,
          pl.BlockSpec((LANES, LANES), lambda i: (0, 0)),
          pl.BlockSpec((1, LANES), lambda i: (0, 0)),
      ],
      out_specs=pspec,
      out_shape=jax.ShapeDtypeStruct((PR, LANES), f32),
  )(s2p, ivp, w_neigh8, b_sage8)


def _tc_combine(s_pn, h, c_term, dip, din, dop, don, wp8, wn8, ws8,
                bpn8, last):
  def body(s_r, h_r, c_r, dip_r, din_r, dop_r, don_r, wp_r, wn_r, ws_r,
           bpn_r, *outs):
    hp = jnp.dot(s_r[0], wp_r[...], preferred_element_type=f32) * dip_r[...]
    hn = jnp.dot(s_r[1], wn_r[...], preferred_element_type=f32) * din_r[...]
    hs = jnp.dot(h_r[...], ws_r[...], preferred_element_type=f32)
    hnew = hp + hn + hs + c_r[...] + bpn_r[...]
    outs[0][...] = hnew
    if not last:
      outs[1][...] = hnew * dop_r[...]
      outs[2][...] = hnew * don_r[...]

  grid = PR // _BN
  pspec = pl.BlockSpec((_BN, LANES), lambda i: (i, 0))
  wspec = pl.BlockSpec((LANES, LANES), lambda i: (0, 0))
  n_out = 1 if last else 3
  return pl.pallas_call(
      body,
      grid=(grid,),
      in_specs=[
          pl.BlockSpec((2, _BN, LANES), lambda i: (0, i, 0)),
          pspec, pspec, pspec, pspec, pspec, pspec, wspec, wspec, wspec,
          pl.BlockSpec((1, LANES), lambda i: (0, 0)),
      ],
      out_specs=[pspec] * n_out,
      out_shape=[jax.ShapeDtypeStruct((PR, LANES), f32)] * n_out,
  )(s_pn, h, c_term, dip, din, dop, don, wp8, wn8, ws8, bpn8)


def _tc_dot(ap, bp, k8t):
  bn = 2048
  grid = PRE // bn

  def body(a_r, b_r, k_r, o_r):
    p = a_r[...] * b_r[...]
    o_r[...] = lax.dot_general(k_r[...], p, (((1,), (1,)), ((), ())),
                               preferred_element_type=f32)

  return pl.pallas_call(
      body,
      grid=(grid,),
      in_specs=[
          pl.BlockSpec((bn, LANES), lambda i: (i, 0)),
          pl.BlockSpec((bn, LANES), lambda i: (i, 0)),
          pl.BlockSpec((8, LANES), lambda i: (0, 0)),
      ],
      out_specs=pl.BlockSpec((8, bn), lambda i: (0, i)),
      out_shape=jax.ShapeDtypeStruct((8, PRE), f32),
  )(ap, bp, k8t)


def _pad2d(a, padval):
  pad = jnp.full((EPAD,), padval, jnp.int32)
  return jnp.concatenate([a, pad]).reshape(EROWS_P, LANES)


def _kron8(w):
  return jnp.kron(jnp.eye(8, dtype=f32), w)


def kernel(x_job, x_worker, edge_precede, edge_next, proc_src, proc_dst,
           W_emb_job, b_emb_job, W_emb_worker, b_emb_worker,
           W_prec, b_prec, W_next, b_next, W_self, W_neigh, b_sage):
  DUMMY = NP

  eps_g = _pad2d(edge_precede[0], 0)
  eps_s = _pad2d(edge_precede[0], DUMMY)
  epd_s = _pad2d(edge_precede[1], DUMMY)
  ens_g = _pad2d(edge_next[0], 0)
  ens_s = _pad2d(edge_next[0], DUMMY)
  end_s = _pad2d(edge_next[1], DUMMY)
  pcs_g = _pad2d(proc_src, 0)
  pcd_s = _pad2d(proc_dst, DUMMY)
  pcd_g = _pad2d(proc_dst, 0)

  x8 = jnp.concatenate(
      [x_job.reshape(N_JOB // 8, 56),
       jnp.zeros(((NP - N_JOB) // 8, 56), f32)])
  xw8 = jnp.concatenate(
      [x_worker.reshape(N_WORKER // 8, 24),
       jnp.zeros(((NWP - N_WORKER) // 8, 24), f32)])
  z16 = jnp.zeros((NP, D), f32)
  ones16 = jnp.ones((LANES, D), f32)
  wej8 = _kron8(W_emb_job)
  wew8 = _kron8(W_emb_worker)
  wp8 = _kron8(W_prec)
  wn8 = _kron8(W_next)
  ws8 = _kron8(W_self)
  wng8 = _kron8(W_neigh)
  bej8 = jnp.tile(b_emb_job, 8).reshape(1, LANES)
  bew8 = jnp.tile(b_emb_worker, 8).reshape(1, LANES)
  bpn8 = jnp.tile(b_prec + b_next, 8).reshape(1, LANES)
  bs8 = jnp.tile(b_sage, 8).reshape(1, LANES)
  k8t = jnp.kron(jnp.eye(8, dtype=f32), jnp.ones((1, 16), f32))

  h0p = _tc_embed(x8, wej8, bej8)
  hwp = _tc_embed(xw8, wew8, bew8)

  counts = _sc_degrees(eps_s, epd_s, ens_s, end_s, pcd_s, z16, ones16)
  counts_p = counts.reshape(NC, 5, PR, LANES)

  hp, hn, dop, don, dip, din, ivp = _tc_prescale(counts_p, h0p)

  hwp, counts_p = lax.optimization_barrier((hwp, counts_p))
  s2 = _sc_sage(hwp.reshape(NWP, D), pcs_g, pcd_s, z16)
  c_term = _tc_sage_c(s2.reshape(NC, PR, LANES), ivp, wng8, bs8)
  hp, s2 = lax.optimization_barrier((hp, s2))

  h = h0p
  for loop in range(NUM_LOOPS):
    last = loop == NUM_LOOPS - 1
    s_pn = _sc_dual_scatter(hp.reshape(NP, D), hn.reshape(NP, D),
                            eps_g, epd_s, ens_g, end_s, z16)
    outs = _tc_combine(s_pn.reshape(NC, PR, LANES), h, c_term,
                       dip, din, dop, don, wp8, wn8, ws8, bpn8, last)
    if last:
      h = outs[0]
    else:
      h, hp, hn = outs

  a_rows, b_rows = _sc_final_gather(hwp.reshape(NWP, D), h.reshape(NP, D),
                                    pcs_g, pcd_g)
  scores_t = _tc_dot(a_rows.reshape(PRE, LANES), b_rows.reshape(PRE, LANES),
                     k8t)
  return scores_t.T.reshape(EP, 1)[:E]


# speedup vs baseline: 17.3972x; 1.1290x over previous
import functools

import jax
import jax.numpy as jnp
from jax import lax
from jax.experimental import pallas as pl
from jax.experimental.pallas import tpu as pltpu
from jax.experimental.pallas import tpu_sc as plsc

N_JOB = 100000
N_WORKER = 10000
E = 1600000
D = 16
NUM_LOOPS = 2

NC = 2
NS = 16
LANES = 128
NP = 100352
PR = NP // 8
NWP = 10240
PRW = NWP // 8
EROWS_P = 12544
EPAD = EROWS_P * LANES - E
EP = EROWS_P * LANES
PRE = EP // 8
CH = 8
ST = NP // NS

_MESH = plsc.VectorSubcoreMesh(core_axis_name="c", subcore_axis_name="s")

f32 = jnp.float32


def _sc_degrees(eps, epd, ens, end_, pcd, z16, ones16):
  @functools.partial(
      pl.kernel,
      mesh=_MESH,
      compiler_params=pltpu.CompilerParams(use_tc_tiling_on_sc=False),
      out_type=jax.ShapeDtypeStruct((NC, 5, NP, D), f32),
      scratch_types=[
          pltpu.VMEM((CH, LANES), jnp.int32),
          pltpu.VMEM((LANES, D), f32),
          pltpu.VMEM_SHARED((NP + 1, D), f32),
          pltpu.SemaphoreType.DMA,
      ],
  )
  def k(eps_h, epd_h, ens_h, end_h, pcd_h, z16_h, ones_h, out_h,
        idx_v, ones_v, acc, sem):
    c = lax.axis_index("c")
    s = lax.axis_index("s")
    pltpu.sync_copy(ones_h, ones_v)

    rows_per_core = EROWS_P // NC
    rows_per_tile = rows_per_core // NS
    nchunks = rows_per_tile // CH
    for r, ih in enumerate((eps_h, epd_h, ens_h, end_h, pcd_h)):
      pltpu.sync_copy(z16_h.at[pl.ds(s * ST, ST)], acc.at[pl.ds(s * ST, ST)])
      plsc.subcore_barrier()

      @pl.loop(0, nchunks)
      def _(g, ih=ih):
        row0 = c * rows_per_core + s * rows_per_tile + g * CH
        pltpu.sync_copy(ih.at[pl.ds(row0, CH)], idx_v)
        descs = [
            pltpu.async_copy(ones_v, acc.at[idx_v.at[b]], sem, add=True)
            for b in range(CH)
        ]
        for d_ in descs:
          d_.wait()

      plsc.subcore_barrier()
      pltpu.sync_copy(acc.at[pl.ds(s * ST, ST)],
                      out_h.at[c, r, pl.ds(s * ST, ST)])

  return k(eps, epd, ens, end_, pcd, z16, ones16)


def _sc_dual_scatter(hp, hn, eps, epd, ens, end_, z16):
  @functools.partial(
      pl.kernel,
      mesh=_MESH,
      compiler_params=pltpu.CompilerParams(use_tc_tiling_on_sc=False),
      out_type=jax.ShapeDtypeStruct((NC, NP, D), f32),
      scratch_types=[
          pltpu.VMEM((2, CH, LANES), jnp.int32),
          pltpu.VMEM((2, CH, LANES), jnp.int32),
          pltpu.VMEM((CH, LANES, D), f32),
          pltpu.VMEM_SHARED((NP + 1, D), f32),
          pltpu.SemaphoreType.DMA,
          pltpu.SemaphoreType.DMA,
          pltpu.SemaphoreType.DMA,
      ],
  )
  def k(hp_h, hn_h, eps_h, epd_h, ens_h, end_h, z16_h, out_h,
        src_v, dst_v, rows_v, acc, gsem, ssem, isem):
    c = lax.axis_index("c")
    s = lax.axis_index("s")
    pltpu.sync_copy(z16_h.at[pl.ds(s * ST, ST)], acc.at[pl.ds(s * ST, ST)])
    plsc.subcore_barrier()

    rows_per_tile = EROWS_P // NS
    nchunks = rows_per_tile // CH

    def run(table_h, src_h, dst_h):
      base = s * rows_per_tile

      def fetch(chunk, p):
        row = base + jnp.minimum(chunk * CH, rows_per_tile - CH)
        return [
            pltpu.async_copy(src_h.at[pl.ds(row, CH)], src_v.at[p], isem),
            pltpu.async_copy(dst_h.at[pl.ds(row, CH)], dst_v.at[p], isem),
        ]

      for d_ in fetch(0, 0):
        d_.wait()

      @pl.loop(0, nchunks // 2)
      def _(t):
        for half in (0, 1):
          chunk = 2 * t + half
          pf = fetch(chunk + 1, 1 - half)
          gd = [
              pltpu.async_copy(table_h.at[src_v.at[half].at[b]],
                               rows_v.at[b], gsem)
              for b in range(CH)
          ]
          sd = []
          for b in range(CH):
            gd[b].wait()
            sd.append(pltpu.async_copy(rows_v.at[b],
                                       acc.at[dst_v.at[half].at[b]],
                                       ssem, add=True))
          for d_ in sd:
            d_.wait()
          for d_ in pf:
            d_.wait()

    @pl.when(c == 0)
    def _():
      run(hp_h, eps_h, epd_h)

    @pl.when(c == 1)
    def _():
      run(hn_h, ens_h, end_h)

    plsc.subcore_barrier()
    pltpu.sync_copy(acc.at[pl.ds(s * ST, ST)],
                    out_h.at[c, pl.ds(s * ST, ST)])

  return k(hp, hn, eps, epd, ens, end_, z16)


def _sc_sage(hw, pcs, pcd, z16):
  @functools.partial(
      pl.kernel,
      mesh=_MESH,
      compiler_params=pltpu.CompilerParams(use_tc_tiling_on_sc=False),
      out_type=jax.ShapeDtypeStruct((NC, NP, D), f32),
      scratch_types=[
          pltpu.VMEM((CH, LANES), jnp.int32),
          pltpu.VMEM((CH, LANES), jnp.int32),
          pltpu.VMEM((CH, LANES, D), f32),
          pltpu.VMEM_SHARED((NP + 1, D), f32),
          pltpu.SemaphoreType.DMA,
          pltpu.SemaphoreType.DMA,
          pltpu.SemaphoreType.DMA,
      ],
  )
  def k(hw_h, src_h, dst_h, z16_h, out_h, src_v, dst_v, rows_v, acc,
        gsem, ssem, isem):
    c = lax.axis_index("c")
    s = lax.axis_index("s")
    pltpu.sync_copy(z16_h.at[pl.ds(s * ST, ST)], acc.at[pl.ds(s * ST, ST)])
    plsc.subcore_barrier()

    rows_per_core = EROWS_P // NC
    rows_per_tile = rows_per_core // NS
    nchunks = rows_per_tile // CH

    @pl.loop(0, nchunks)
    def _(g):
      row0 = c * rows_per_core + s * rows_per_tile + g * CH
      fi = [pltpu.async_copy(src_h.at[pl.ds(row0, CH)], src_v, isem),
            pltpu.async_copy(dst_h.at[pl.ds(row0, CH)], dst_v, isem)]
      for d_ in fi:
        d_.wait()
      gd = [
          pltpu.async_copy(hw_h.at[src_v.at[b]], rows_v.at[b], gsem)
          for b in range(CH)
      ]
      sd = []
      for b in range(CH):
        gd[b].wait()
        sd.append(pltpu.async_copy(rows_v.at[b], acc.at[dst_v.at[b]],
                                   ssem, add=True))
      for d_ in sd:
        d_.wait()

    plsc.subcore_barrier()
    pltpu.sync_copy(acc.at[pl.ds(s * ST, ST)],
                    out_h.at[c, pl.ds(s * ST, ST)])

  return k(hw, pcs, pcd, z16)


def _sc_final_gather(hw, h2, pcs, pcdg):
  @functools.partial(
      pl.kernel,
      mesh=_MESH,
      compiler_params=pltpu.CompilerParams(use_tc_tiling_on_sc=False),
      out_type=[
          jax.ShapeDtypeStruct((EROWS_P, LANES, D), f32),
          jax.ShapeDtypeStruct((EROWS_P, LANES, D), f32),
      ],
      scratch_types=[
          pltpu.VMEM((CH, LANES), jnp.int32),
          pltpu.VMEM((CH, LANES), jnp.int32),
          pltpu.VMEM((CH, LANES, D), f32),
          pltpu.VMEM((CH, LANES, D), f32),
          pltpu.SemaphoreType.DMA,
          pltpu.SemaphoreType.DMA,
          pltpu.SemaphoreType.DMA,
      ],
  )
  def k(hw_h, h2_h, src_h, dst_h, a_h, b_h, src_v, dst_v, arows_v, brows_v,
        gsem, ssem, isem):
    c = lax.axis_index("c")
    s = lax.axis_index("s")
    rows_per_core = EROWS_P // NC
    rows_per_tile = rows_per_core // NS
    nchunks = rows_per_tile // CH

    @pl.loop(0, nchunks)
    def _(g):
      row0 = c * rows_per_core + s * rows_per_tile + g * CH
      fi = [pltpu.async_copy(src_h.at[pl.ds(row0, CH)], src_v, isem),
            pltpu.async_copy(dst_h.at[pl.ds(row0, CH)], dst_v, isem)]
      for d_ in fi:
        d_.wait()
      ga = [
          pltpu.async_copy(hw_h.at[src_v.at[b]], arows_v.at[b], gsem)
          for b in range(CH)
      ]
      gb = [
          pltpu.async_copy(h2_h.at[dst_v.at[b]], brows_v.at[b], gsem)
          for b in range(CH)
      ]
      for d_ in ga:
        d_.wait()
      sa = pltpu.async_copy(arows_v, a_h.at[pl.ds(row0, CH)], ssem)
      for d_ in gb:
        d_.wait()
      sb = pltpu.async_copy(brows_v, b_h.at[pl.ds(row0, CH)], ssem)
      sa.wait()
      sb.wait()

  return k(hw, h2, pcs, pcdg)


_BN = 784


def _tc_embed(x8, w8, b8):
  rows, din8 = x8.shape
  bn = rows // 16 if rows % 16 == 0 else rows
  grid = rows // bn

  def body(x_r, w_r, b_r, o_r):
    o_r[...] = jnp.dot(x_r[...], w_r[...],
                       preferred_element_type=f32) + b_r[...]

  return pl.pallas_call(
      body,
      grid=(grid,),
      in_specs=[
          pl.BlockSpec((bn, din8), lambda i: (i, 0)),
          pl.BlockSpec((din8, LANES), lambda i: (0, 0)),
          pl.BlockSpec((1, LANES), lambda i: (0, 0)),
      ],
      out_specs=pl.BlockSpec((bn, LANES), lambda i: (i, 0)),
      out_shape=jax.ShapeDtypeStruct((rows, LANES), f32),
  )(x8, w8, b8)


def _tc_prescale(counts_p, h0p):
  def body(c_r, h_r, hp_r, hn_r, dop_r, don_r, dip_r, din_r, ivp_r):
    ctot = c_r[0] + c_r[1]
    dop = lax.rsqrt(jnp.maximum(ctot[0], 1.0))
    dip = lax.rsqrt(jnp.maximum(ctot[1], 1.0))
    don = lax.rsqrt(jnp.maximum(ctot[2], 1.0))
    din = lax.rsqrt(jnp.maximum(ctot[3], 1.0))
    ivp = 1.0 / jnp.maximum(ctot[4], 1.0)
    h = h_r[...]
    hp_r[...] = h * dop
    hn_r[...] = h * don
    dop_r[...] = dop
    don_r[...] = don
    dip_r[...] = dip
    din_r[...] = din
    ivp_r[...] = ivp

  grid = PR // _BN
  pspec = pl.BlockSpec((_BN, LANES), lambda i: (i, 0))
  return pl.pallas_call(
      body,
      grid=(grid,),
      in_specs=[
          pl.BlockSpec((2, 5, _BN, LANES), lambda i: (0, 0, i, 0)),
          pspec,
      ],
      out_specs=[pspec] * 7,
      out_shape=[jax.ShapeDtypeStruct((PR, LANES), f32)] * 7,
  )(counts_p, h0p)


def _tc_sage_c(s2p, ivp, w_neigh8, b_sage8):
  def body(s_r, ivp_r, w_r, b_r, o_r):
    mean = (s_r[0] + s_r[1]) * ivp_r[...]
    o_r[...] = jnp.dot(mean, w_r[...], preferred_element_type=f32) + b_r[...]

  grid = PR // _BN
  pspec = pl.BlockSpec((_BN, LANES), lambda i: (i, 0))
  return pl.pallas_call(
      body,
      grid=(grid,),
      in_specs=[
          pl.BlockSpec((2, _BN, LANES), lambda i: (0, i, 0)),
          pspec,
          pl.BlockSpec((LANES, LANES), lambda i: (0, 0)),
          pl.BlockSpec((1, LANES), lambda i: (0, 0)),
      ],
      out_specs=pspec,
      out_shape=jax.ShapeDtypeStruct((PR, LANES), f32),
  )(s2p, ivp, w_neigh8, b_sage8)


def _tc_combine(s_pn, h, c_term, dip, din, dop, don, wp8, wn8, ws8,
                bpn8, last):
  def body(s_r, h_r, c_r, dip_r, din_r, dop_r, don_r, wp_r, wn_r, ws_r,
           bpn_r, *outs):
    hp = jnp.dot(s_r[0], wp_r[...], preferred_element_type=f32) * dip_r[...]
    hn = jnp.dot(s_r[1], wn_r[...], preferred_element_type=f32) * din_r[...]
    hs = jnp.dot(h_r[...], ws_r[...], preferred_element_type=f32)
    hnew = hp + hn + hs + c_r[...] + bpn_r[...]
    outs[0][...] = hnew
    if not last:
      outs[1][...] = hnew * dop_r[...]
      outs[2][...] = hnew * don_r[...]

  grid = PR // _BN
  pspec = pl.BlockSpec((_BN, LANES), lambda i: (i, 0))
  wspec = pl.BlockSpec((LANES, LANES), lambda i: (0, 0))
  n_out = 1 if last else 3
  return pl.pallas_call(
      body,
      grid=(grid,),
      in_specs=[
          pl.BlockSpec((2, _BN, LANES), lambda i: (0, i, 0)),
          pspec, pspec, pspec, pspec, pspec, pspec, wspec, wspec, wspec,
          pl.BlockSpec((1, LANES), lambda i: (0, 0)),
      ],
      out_specs=[pspec] * n_out,
      out_shape=[jax.ShapeDtypeStruct((PR, LANES), f32)] * n_out,
  )(s_pn, h, c_term, dip, din, dop, don, wp8, wn8, ws8, bpn8)


def _tc_dot(ap, bp, k8t):
  bn = 2048
  grid = PRE // bn

  def body(a_r, b_r, k_r, o_r):
    p = a_r[...] * b_r[...]
    o_r[...] = lax.dot_general(k_r[...], p, (((1,), (1,)), ((), ())),
                               preferred_element_type=f32)

  return pl.pallas_call(
      body,
      grid=(grid,),
      in_specs=[
          pl.BlockSpec((bn, LANES), lambda i: (i, 0)),
          pl.BlockSpec((bn, LANES), lambda i: (i, 0)),
          pl.BlockSpec((8, LANES), lambda i: (0, 0)),
      ],
      out_specs=pl.BlockSpec((8, bn), lambda i: (0, i)),
      out_shape=jax.ShapeDtypeStruct((8, PRE), f32),
  )(ap, bp, k8t)


def _pad2d(a, padval):
  pad = jnp.full((EPAD,), padval, jnp.int32)
  return jnp.concatenate([a, pad]).reshape(EROWS_P, LANES)


def _kron8(w):
  return jnp.kron(jnp.eye(8, dtype=f32), w)


def kernel(x_job, x_worker, edge_precede, edge_next, proc_src, proc_dst,
           W_emb_job, b_emb_job, W_emb_worker, b_emb_worker,
           W_prec, b_prec, W_next, b_next, W_self, W_neigh, b_sage):
  DUMMY = NP

  eps_g = _pad2d(edge_precede[0], 0)
  eps_s = _pad2d(edge_precede[0], DUMMY)
  epd_s = _pad2d(edge_precede[1], DUMMY)
  ens_g = _pad2d(edge_next[0], 0)
  ens_s = _pad2d(edge_next[0], DUMMY)
  end_s = _pad2d(edge_next[1], DUMMY)
  pcs_g = _pad2d(proc_src, 0)
  pcd_s = _pad2d(proc_dst, DUMMY)
  pcd_g = _pad2d(proc_dst, 0)

  x8 = jnp.concatenate(
      [x_job.reshape(N_JOB // 8, 56),
       jnp.zeros(((NP - N_JOB) // 8, 56), f32)])
  xw8 = jnp.concatenate(
      [x_worker.reshape(N_WORKER // 8, 24),
       jnp.zeros(((NWP - N_WORKER) // 8, 24), f32)])
  z16 = jnp.zeros((NP, D), f32)
  ones16 = jnp.ones((LANES, D), f32)
  wej8 = _kron8(W_emb_job)
  wew8 = _kron8(W_emb_worker)
  wp8 = _kron8(W_prec)
  wn8 = _kron8(W_next)
  ws8 = _kron8(W_self)
  wng8 = _kron8(W_neigh)
  bej8 = jnp.tile(b_emb_job, 8).reshape(1, LANES)
  bew8 = jnp.tile(b_emb_worker, 8).reshape(1, LANES)
  bpn8 = jnp.tile(b_prec + b_next, 8).reshape(1, LANES)
  bs8 = jnp.tile(b_sage, 8).reshape(1, LANES)
  k8t = jnp.kron(jnp.eye(8, dtype=f32), jnp.ones((1, 16), f32))

  h0p = _tc_embed(x8, wej8, bej8)
  hwp = _tc_embed(xw8, wew8, bew8)

  counts = _sc_degrees(eps_s, epd_s, ens_s, end_s, pcd_s, z16, ones16)
  counts_p = counts.reshape(NC, 5, PR, LANES)

  hp, hn, dop, don, dip, din, ivp = _tc_prescale(counts_p, h0p)

  hwp, counts_p = lax.optimization_barrier((hwp, counts_p))
  s2 = _sc_sage(hwp.reshape(NWP, D), pcs_g, pcd_s, z16)
  c_term = _tc_sage_c(s2.reshape(NC, PR, LANES), ivp, wng8, bs8)
  hp, s2 = lax.optimization_barrier((hp, s2))

  h = h0p
  for loop in range(NUM_LOOPS):
    last = loop == NUM_LOOPS - 1
    s_pn = _sc_dual_scatter(hp.reshape(NP, D), hn.reshape(NP, D),
                            eps_g, epd_s, ens_g, end_s, z16)
    outs = _tc_combine(s_pn.reshape(NC, PR, LANES), h, c_term,
                       dip, din, dop, don, wp8, wn8, ws8, bpn8, last)
    if last:
      h = outs[0]
    else:
      h, hp, hn = outs

  a_rows, b_rows = _sc_final_gather(hwp.reshape(NWP, D), h.reshape(NP, D),
                                    pcs_g, pcd_g)
  scores_t = _tc_dot(a_rows.reshape(PRE, LANES), b_rows.reshape(PRE, LANES),
                     k8t)
  return scores_t.T.reshape(EP, 1)[:E]
